# Initial kernel scaffold; baseline (speedup 1.0000x reference)
import functools

import jax
import jax.numpy as jnp
from jax import lax
from jax.experimental import pallas as pl
from jax.experimental.pallas import tpu as pltpu
from jax.experimental.pallas import tpu_sc as plsc

N_NODES_C = 10000
N_EDGES_C = 320000
D_FEAT_C = 128

NUM_CORES = 2
NUM_SUBCORES = 16
NUM_WORKERS = NUM_CORES * NUM_SUBCORES
E_PER_TILE = N_EDGES_C // NUM_WORKERS
CHUNK = 80
N_CHUNKS = E_PER_TILE // CHUNK
ROWS_PER_TILE = N_NODES_C // NUM_SUBCORES
ZCHUNK = 125
LANES = 16
D_SLICES = D_FEAT_C // LANES


def _sc_partials(x, row, col, val):
  mesh = plsc.VectorSubcoreMesh(core_axis_name="c", subcore_axis_name="s")

  @functools.partial(
      pl.kernel,
      mesh=mesh,
      out_type=jax.ShapeDtypeStruct((NUM_CORES, N_NODES_C, D_FEAT_C),
                                    jnp.float32),
      scratch_types=[
          pltpu.VMEM((CHUNK,), jnp.int32),
          pltpu.VMEM((CHUNK,), jnp.int32),
          pltpu.VMEM((CHUNK,), jnp.float32),
          pltpu.VMEM((CHUNK, D_FEAT_C), jnp.float32),
          pltpu.VMEM((ZCHUNK, D_FEAT_C), jnp.float32),
          pltpu.VMEM_SHARED((N_NODES_C, D_FEAT_C), jnp.float32),
          pltpu.SemaphoreType.DMA,
      ],
  )
  def k(x_hbm, row_hbm, col_hbm, val_hbm, out_hbm,
        colv, rowv, valv, rows, zbuf, ypar, sem):
    c = lax.axis_index("c")
    s = lax.axis_index("s")
    wid = c * NUM_SUBCORES + s

    zero = jnp.zeros((LANES,), jnp.float32)

    def zfill(i, _):
      for j in range(D_SLICES):
        zbuf[i, pl.ds(j * LANES, LANES)] = zero
      return 0

    lax.fori_loop(0, ZCHUNK, zfill, 0)
    for zi in range(ROWS_PER_TILE // ZCHUNK):
      pltpu.sync_copy(zbuf,
                      ypar.at[pl.ds(s * ROWS_PER_TILE + zi * ZCHUNK, ZCHUNK)])
    plsc.subcore_barrier()

    ebase = wid * E_PER_TILE

    def body(i, _):
      base = ebase + i * CHUNK
      pltpu.sync_copy(col_hbm.at[pl.ds(base, CHUNK)], colv)
      pltpu.sync_copy(row_hbm.at[pl.ds(base, CHUNK)], rowv)
      pltpu.sync_copy(val_hbm.at[pl.ds(base, CHUNK)], valv)
      pltpu.async_copy(x_hbm.at[colv], rows, sem).wait()

      def scale(e, _):
        v = valv[e]
        for j in range(D_SLICES):
          sl = pl.ds(j * LANES, LANES)
          rows[e, sl] = rows[e, sl] * v
        return 0

      lax.fori_loop(0, CHUNK, scale, 0)
      pltpu.sync_copy(rows, ypar.at[rowv], add=True)
      return 0

    lax.fori_loop(0, N_CHUNKS, body, 0)
    plsc.subcore_barrier()

    pltpu.sync_copy(ypar.at[pl.ds(s * ROWS_PER_TILE, ROWS_PER_TILE)],
                    out_hbm.at[c, pl.ds(s * ROWS_PER_TILE, ROWS_PER_TILE)])

  return k(x, row, col, val)


def _combine(partials):
  def body(p_ref, o_ref):
    o_ref[...] = p_ref[0] + p_ref[1]

  blk = 1000
  return pl.pallas_call(
      body,
      grid=(N_NODES_C // blk,),
      in_specs=[pl.BlockSpec((NUM_CORES, blk, D_FEAT_C),
                             lambda i: (0, i, 0))],
      out_specs=pl.BlockSpec((blk, D_FEAT_C), lambda i: (i, 0)),
      out_shape=jax.ShapeDtypeStruct((N_NODES_C, D_FEAT_C), jnp.float32),
  )(partials)


def kernel(x, edge_index, edge_values):
  row = edge_index[0]
  col = edge_index[1]
  partials = _sc_partials(x, row, col, edge_values)
  return _combine(partials)


# speedup vs baseline: 4.5423x; 4.5423x over previous
import functools

import jax
import jax.numpy as jnp
from jax import lax
from jax.experimental import pallas as pl
from jax.experimental.pallas import tpu as pltpu
from jax.experimental.pallas import tpu_sc as plsc

N_NODES_C = 10000
N_EDGES_C = 320000
D_FEAT_C = 128

NUM_CORES = 2
NUM_SUBCORES = 16
NUM_WORKERS = NUM_CORES * NUM_SUBCORES
E_PER_TILE = N_EDGES_C // NUM_WORKERS
CHUNK = 80
N_CHUNKS = E_PER_TILE // CHUNK
ROWS_PER_TILE = N_NODES_C // NUM_SUBCORES
ZCHUNK = 125
LANES = 16
D_SLICES = D_FEAT_C // LANES


def _sc_partials(x, row, col, val):
  mesh = plsc.VectorSubcoreMesh(core_axis_name="c", subcore_axis_name="s")

  @functools.partial(
      pl.kernel,
      mesh=mesh,
      out_type=jax.ShapeDtypeStruct((NUM_CORES, N_NODES_C, D_FEAT_C),
                                    jnp.float32),
      scratch_types=[
          pltpu.VMEM((CHUNK,), jnp.int32),
          pltpu.VMEM((CHUNK,), jnp.int32),
          pltpu.VMEM((CHUNK,), jnp.float32),
          pltpu.VMEM((CHUNK, D_FEAT_C), jnp.float32),
          pltpu.VMEM((ZCHUNK, D_FEAT_C), jnp.float32),
          pltpu.VMEM_SHARED((N_NODES_C, D_FEAT_C), jnp.float32),
          pltpu.SemaphoreType.DMA,
      ],
  )
  def k(x_hbm, row_hbm, col_hbm, val_hbm, out_hbm,
        colv, rowv, valv, rows, zbuf, ypar, sem):
    c = lax.axis_index("c")
    s = lax.axis_index("s")
    wid = c * NUM_SUBCORES + s

    zero = jnp.zeros((LANES,), jnp.float32)

    def zfill(i, _):
      for j in range(D_SLICES):
        zbuf[i, pl.ds(j * LANES, LANES)] = zero
      return 0

    lax.fori_loop(0, ZCHUNK, zfill, 0)
    for zi in range(ROWS_PER_TILE // ZCHUNK):
      pltpu.sync_copy(zbuf,
                      ypar.at[pl.ds(s * ROWS_PER_TILE + zi * ZCHUNK, ZCHUNK)])
    plsc.subcore_barrier()

    ebase = wid * E_PER_TILE

    def body(i, _):
      base = ebase + i * CHUNK
      pltpu.sync_copy(col_hbm.at[pl.ds(base, CHUNK)], colv)
      pltpu.sync_copy(row_hbm.at[pl.ds(base, CHUNK)], rowv)
      pltpu.sync_copy(val_hbm.at[pl.ds(base, CHUNK)], valv)
      pltpu.async_copy(x_hbm.at[colv], rows, sem).wait()

      def scale16(g, _):
        vv = valv[pl.ds(g * LANES, LANES)]
        for l in range(LANES):
          v = vv[l]
          e = g * LANES + l
          for j in range(D_SLICES):
            sl = pl.ds(j * LANES, LANES)
            rows[e, sl] = rows[e, sl] * v
        return 0

      lax.fori_loop(0, CHUNK // LANES, scale16, 0)
      pltpu.sync_copy(rows, ypar.at[rowv], add=True)
      return 0

    lax.fori_loop(0, N_CHUNKS, body, 0)
    plsc.subcore_barrier()

    W = 624
    pltpu.sync_copy(ypar.at[pl.ds(s * W, W)],
                    out_hbm.at[c, pl.ds(s * W, W)])

    @pl.when(s == 0)
    def _tail():
      tail = N_NODES_C - NUM_SUBCORES * W
      pltpu.sync_copy(ypar.at[pl.ds(NUM_SUBCORES * W, tail)],
                      out_hbm.at[c, pl.ds(NUM_SUBCORES * W, tail)])

  return k(x, row, col, val)


def _combine(partials):
  def body(p_ref, o_ref):
    o_ref[...] = p_ref[0] + p_ref[1]

  blk = 1000
  return pl.pallas_call(
      body,
      grid=(N_NODES_C // blk,),
      in_specs=[pl.BlockSpec((NUM_CORES, blk, D_FEAT_C),
                             lambda i: (0, i, 0))],
      out_specs=pl.BlockSpec((blk, D_FEAT_C), lambda i: (i, 0)),
      out_shape=jax.ShapeDtypeStruct((N_NODES_C, D_FEAT_C), jnp.float32),
  )(partials)


def kernel(x, edge_index, edge_values):
  row = edge_index[0]
  col = edge_index[1]
  partials = _sc_partials(x, row, col, edge_values)
  return _combine(partials)


# speedup vs baseline: 10.3025x; 2.2681x over previous
import functools

import jax
import jax.numpy as jnp
from jax import lax
from jax.experimental import pallas as pl
from jax.experimental.pallas import tpu as pltpu
from jax.experimental.pallas import tpu_sc as plsc

N_NODES_C = 10000
N_EDGES_C = 320000
D_FEAT_C = 128

NUM_CORES = 2
NUM_SUBCORES = 16
NUM_WORKERS = NUM_CORES * NUM_SUBCORES
E_PER_TILE = N_EDGES_C // NUM_WORKERS
CHUNK = 80
N_CHUNKS = E_PER_TILE // CHUNK
ROWS_PER_TILE = N_NODES_C // NUM_SUBCORES
LANES = 16
D_SLICES = D_FEAT_C // LANES
EDGE_GROUPS = CHUNK // LANES


def _sc_partials(x, row3, col3, val3):
  mesh = plsc.VectorSubcoreMesh(core_axis_name="c", subcore_axis_name="s")

  @functools.partial(
      pl.kernel,
      mesh=mesh,
      out_type=jax.ShapeDtypeStruct((NUM_CORES, N_NODES_C, D_FEAT_C),
                                    jnp.float32),
      scratch_types=[
          pltpu.VMEM((E_PER_TILE,), jnp.int32),
          pltpu.VMEM((E_PER_TILE,), jnp.float32),
          pltpu.VMEM((1, CHUNK), jnp.int32),
          pltpu.VMEM((1, CHUNK), jnp.int32),
          pltpu.VMEM((CHUNK, D_FEAT_C), jnp.float32),
          pltpu.VMEM((CHUNK, D_FEAT_C), jnp.float32),
          pltpu.VMEM_SHARED((N_NODES_C, D_FEAT_C), jnp.float32),
          pltpu.SemaphoreType.DMA,
          pltpu.SemaphoreType.DMA,
          pltpu.SemaphoreType.DMA,
      ],
  )
  def k(x_hbm, row_hbm, col_hbm, val_hbm, out_hbm,
        colv, valv, rowv0, rowv1, rows0, rows1, ypar, psem, gsem0, gsem1):
    c = lax.axis_index("c")
    s = lax.axis_index("s")
    wid = c * NUM_SUBCORES + s

    h_col = pltpu.async_copy(col_hbm.at[wid], colv, psem)
    h_val = pltpu.async_copy(val_hbm.at[wid], valv, psem)

    zero = jnp.zeros((LANES,), jnp.float32)

    def zfill(i, _):
      for j in range(D_SLICES):
        rows0[i, pl.ds(j * LANES, LANES)] = zero
      return 0

    lax.fori_loop(0, CHUNK, zfill, 0)
    zbase = s * ROWS_PER_TILE
    for zi in range(ROWS_PER_TILE // CHUNK):
      pltpu.sync_copy(rows0, ypar.at[pl.ds(zbase + zi * CHUNK, CHUNK)])
    ztail = ROWS_PER_TILE % CHUNK
    pltpu.sync_copy(rows0.at[pl.ds(0, ztail)],
                    ypar.at[pl.ds(zbase + ROWS_PER_TILE - ztail, ztail)])
    h_col.wait()
    h_val.wait()
    plsc.subcore_barrier()

    def issue(j, rows_buf, rowv_buf, sem):
      pltpu.async_copy(x_hbm.at[colv.at[pl.ds(j * CHUNK, CHUNK)]],
                       rows_buf, sem)
      pltpu.async_copy(row_hbm.at[wid, pl.ds(j, 1)], rowv_buf, sem)

    def drain(j, rows_buf, rowv_buf, sem):
      pltpu.make_async_copy(x_hbm.at[colv.at[pl.ds(j * CHUNK, CHUNK)]],
                            rows_buf, sem).wait()
      pltpu.make_async_copy(row_hbm.at[wid, pl.ds(j, 1)], rowv_buf,
                            sem).wait()

    def scale(buf, j):
      def scale16(g, _):
        vv = valv[pl.ds(j * CHUNK + g * LANES, LANES)]
        for l in range(LANES):
          v = vv[l]
          e = g * LANES + l
          for d in range(D_SLICES):
            sl = pl.ds(d * LANES, LANES)
            buf[e, sl] = buf[e, sl] * v
        return 0

      lax.fori_loop(0, EDGE_GROUPS, scale16, 0)

    def phase(j, cur, rcur, csem, nxt, rnxt, nsem):
      drain(j, cur, rcur, csem)

      @pl.when(j + 1 < N_CHUNKS)
      def _():
        issue(j + 1, nxt, rnxt, nsem)

      scale(cur, j)
      pltpu.sync_copy(cur, ypar.at[rcur.at[0]], add=True)

    issue(0, rows0, rowv0, gsem0)

    def body(j, _):
      @pl.when(j % 2 == 0)
      def _():
        phase(j, rows0, rowv0, gsem0, rows1, rowv1, gsem1)

      @pl.when(j % 2 == 1)
      def _():
        phase(j, rows1, rowv1, gsem1, rows0, rowv0, gsem0)

      return 0

    lax.fori_loop(0, N_CHUNKS, body, 0)
    plsc.subcore_barrier()

    W = 624
    pltpu.sync_copy(ypar.at[pl.ds(s * W, W)],
                    out_hbm.at[c, pl.ds(s * W, W)])

    @pl.when(s == 0)
    def _tail():
      tail = N_NODES_C - NUM_SUBCORES * W
      pltpu.sync_copy(ypar.at[pl.ds(NUM_SUBCORES * W, tail)],
                      out_hbm.at[c, pl.ds(NUM_SUBCORES * W, tail)])

  return k(x, row3, col3, val3)


def _combine(partials):
  def body(p_ref, o_ref):
    o_ref[...] = p_ref[0] + p_ref[1]

  blk = 1000
  return pl.pallas_call(
      body,
      grid=(N_NODES_C // blk,),
      in_specs=[pl.BlockSpec((NUM_CORES, blk, D_FEAT_C),
                             lambda i: (0, i, 0))],
      out_specs=pl.BlockSpec((blk, D_FEAT_C), lambda i: (i, 0)),
      out_shape=jax.ShapeDtypeStruct((N_NODES_C, D_FEAT_C), jnp.float32),
  )(partials)


def kernel(x, edge_index, edge_values):
  row3 = edge_index[0].reshape(NUM_WORKERS, N_CHUNKS, CHUNK)
  col3 = edge_index[1].reshape(NUM_WORKERS, E_PER_TILE)
  val3 = edge_values.reshape(NUM_WORKERS, E_PER_TILE)
  partials = _sc_partials(x, row3, col3, val3)
  return _combine(partials)
